# Initial kernel scaffold; baseline (speedup 1.0000x reference)
import jax
import jax.numpy as jnp
from jax.experimental import pallas as pl


def kernel(x, table, W, b):
    raise NotImplementedError("write your pallas kernel here")


# speedup vs baseline: 1.7681x; 1.7681x over previous
import functools

import jax
import jax.numpy as jnp
from jax import lax
from jax.experimental import pallas as pl
from jax.experimental.pallas import tpu as pltpu
from jax.experimental.pallas import tpu_sc as plsc


def _sc_gather(table, idx):
    V, D = table.shape
    (B,) = idx.shape
    info = plsc.get_sparse_core_info()
    NC, NS = info.num_cores, info.num_subcores
    NW = NC * NS
    assert B % NW == 0
    b_per_w = B // NW
    GC = 128
    SUPER = 512
    assert b_per_w % SUPER == 0 and SUPER % GC == 0
    n_super = b_per_w // SUPER
    n_g = SUPER // GC

    mesh = plsc.VectorSubcoreMesh(core_axis_name="c", subcore_axis_name="s")

    @functools.partial(
        pl.kernel,
        mesh=mesh,
        compiler_params=pltpu.CompilerParams(use_tc_tiling_on_sc=False),
        out_type=jax.ShapeDtypeStruct((B, D), jnp.float32),
        scratch_types=[
            pltpu.VMEM((b_per_w,), jnp.int32),
            pltpu.VMEM((SUPER, D), jnp.float32),
            pltpu.SemaphoreType.DMA,
        ],
    )
    def k(table_hbm, idx_hbm, out_hbm, idx_v, rows_v, gsem):
        wid = lax.axis_index("s") * NC + lax.axis_index("c")
        base = wid * b_per_w
        pltpu.sync_copy(idx_hbm.at[pl.ds(base, b_per_w)], idx_v)

        def body(si, carry):
            descs = []
            for j in range(n_g):
                d = pltpu.async_copy(
                    table_hbm.at[idx_v.at[pl.ds(si * SUPER + j * GC, GC)]],
                    rows_v.at[pl.ds(j * GC, GC)],
                    gsem,
                )
                descs.append(d)
            for d in descs:
                d.wait()
            pltpu.sync_copy(rows_v, out_hbm.at[pl.ds(base + si * SUPER, SUPER)])
            return carry

        lax.fori_loop(0, n_super, body, 0)

    return k(table, idx)


_SQRT_HALF = 0.7071067811865476


def _adapter_body(x_ref, w_ref, b_ref, o_ref):
    h = jnp.dot(x_ref[...], w_ref[...], preferred_element_type=jnp.float32)
    h = h + b_ref[...]
    o_ref[...] = h * 0.5 * (1.0 + lax.erf(h * _SQRT_HALF))


def _tc_adapter(g2, W2, b2):
    R, C = g2.shape
    BLK = 2048
    assert R % BLK == 0
    return pl.pallas_call(
        _adapter_body,
        grid=(R // BLK,),
        in_specs=[
            pl.BlockSpec((BLK, C), lambda i: (i, 0)),
            pl.BlockSpec((C, C), lambda i: (0, 0)),
            pl.BlockSpec((1, C), lambda i: (0, 0)),
        ],
        out_specs=pl.BlockSpec((BLK, C), lambda i: (i, 0)),
        out_shape=jax.ShapeDtypeStruct((R, C), jnp.float32),
    )(g2, W2, b2)


def kernel(x, table, W, b):
    Bt, S = x.shape
    V, D = table.shape
    Bf = Bt * S
    idx = x.reshape(Bf).astype(jnp.int32)
    gathered = _sc_gather(table, idx)
    g2 = gathered.reshape(Bf // 2, 2 * D)
    Wt = W.T
    W2 = (
        jnp.zeros((2 * D, 2 * D), jnp.float32)
        .at[:D, :D].set(Wt)
        .at[D:, D:].set(Wt)
    )
    b2 = jnp.concatenate([b, b]).reshape(1, 2 * D)
    out = _tc_adapter(g2, W2, b2)
    return out.reshape(Bt, S, D)
